# baseline (speedup 1.0000x reference)
import functools

import jax
import jax.numpy as jnp
from jax import lax
from jax.experimental import pallas as pl
from jax.experimental.pallas import tpu as pltpu
from jax.experimental.pallas import tpu_sc as plsc

_N = 10000
_E = 160000
_HEADS = 6
_NCORES = 2
_NSUB = 16
_NW = _NCORES * _NSUB
_C = 128
_NCHUNKS = _E // _C
_CHUNK_BASE = _NCHUNKS // _NW
_CHUNK_REM = _NCHUNKS - _CHUNK_BASE * _NW
_NPAD = 10240
_RPT = _NPAD // _NSUB
_ZR = 128
_NH = _NPAD // 2
_NHT = _NH + _ZR
_RPTD = _NH // _NSUB
_BN = 1000
_NBLK = _N // _BN


def _make_edge_kernel(hf2, filt, q):
    groups = hf2 // 16
    gpf = filt // 16
    lanes0 = [((2 * q + 0) * hf2 // 16 + g) // gpf for g in range(groups)]
    lanes1 = [((2 * q + 1) * hf2 // 16 + g) // gpf for g in range(groups)]
    mesh = plsc.VectorSubcoreMesh(core_axis_name="c", subcore_axis_name="s",
                                  num_cores=_NCORES, num_subcores=_NSUB)

    out_type = (
        jax.ShapeDtypeStruct((_NCORES, _NPAD, hf2), jnp.float32),
        jax.ShapeDtypeStruct((_NPAD, 16), jnp.float32),
    )
    scratch_types = [
        pltpu.VMEM((2, _C), jnp.int32),
        pltpu.VMEM((2, _C), jnp.int32),
        pltpu.VMEM((2, _C), jnp.int32),
        pltpu.VMEM((2, _C), jnp.int32),
        pltpu.VMEM((2, _C, 16), jnp.float32),
        pltpu.VMEM((2, _C, 16), jnp.float32),
        pltpu.VMEM((_C, 16), jnp.float32),
        pltpu.VMEM((2, _C, hf2), jnp.float32),
        pltpu.VMEM((2, 16), jnp.float32),
        pltpu.VMEM((_ZR, hf2), jnp.float32),
        pltpu.VMEM((_ZR, 16), jnp.float32),
        pltpu.VMEM_SHARED((_NPAD, hf2), jnp.float32),
        pltpu.VMEM_SHARED((_NHT, 16), jnp.float32),
        pltpu.SemaphoreType.DMA((2,)),
    ]

    @functools.partial(
        pl.kernel, mesh=mesh, out_type=out_type, scratch_types=scratch_types,
        compiler_params=pltpu.CompilerParams(needs_layout_passes=False,
                                             use_tc_tiling_on_sc=False))
    def edge_kernel(src_hbm, dst_hbm, wh_hbm, s_hbm, d_hbm, m_hbm,
                    num_out, den_out,
                    src_v, dst_v, idx2_v, idxd_v, s_rows, d_rows, ex_v, wh_v,
                    m_v, zb, zbd, num_sh, den_sh, sem):
        c = lax.axis_index("c")
        sid = lax.axis_index("s")
        zvec = jnp.zeros((16,), jnp.float32)

        def zrow(r, carry):
            for g in range(groups):
                zb[r, pl.ds(g * 16, 16)] = zvec
            zbd[r, :] = zvec
            return carry
        lax.fori_loop(0, _ZR, zrow, 0)

        r0 = sid * _RPT
        for k in range(_RPT // _ZR):
            pltpu.sync_copy(zb, num_sh.at[pl.ds(r0 + k * _ZR, _ZR)])
        rd0 = sid * _RPTD
        for k in range(_RPTD // _ZR):
            pltpu.sync_copy(zbd, den_sh.at[pl.ds(rd0 + k * _ZR, _ZR)])
        pltpu.sync_copy(zbd.at[pl.ds(0, 8)], den_sh.at[pl.ds(_NH + sid * 8, 8)])
        pltpu.sync_copy(m_hbm, m_v)
        plsc.subcore_barrier()

        msum = m_v[0, :] + m_v[1, :]
        mvec = jnp.maximum(msum, 0.2 * msum)
        lanes = jax.lax.iota(jnp.int32, 16)
        trash = lanes % 8 + (_NH + sid * 8)
        coff = c * _N
        dlo = c * _NH

        w = sid * _NCORES + c
        start = w * _CHUNK_BASE + jnp.minimum(w, _CHUNK_REM)
        nk = _CHUNK_BASE + jnp.where(w < _CHUNK_REM, 1, 0)

        def fire(bb, k):
            base = k * _C
            pltpu.sync_copy(src_hbm.at[pl.ds(base, _C)], src_v.at[bb])
            pltpu.sync_copy(dst_hbm.at[pl.ds(base, _C)], dst_v.at[bb])

            def addl(j, carry2):
                sl = pl.ds(j * 16, 16)
                idx2_v[bb, sl] = src_v[bb, sl] + coff
                t = dst_v[bb, sl] - dlo
                oob = (t < 0) | (t >= _NH)
                idxd_v[bb, sl] = jnp.where(oob, trash, t)
                return carry2
            lax.fori_loop(0, _C // 16, addl, 0)
            pltpu.async_copy(s_hbm.at[src_v.at[bb]], s_rows.at[bb], sem.at[bb])
            pltpu.async_copy(d_hbm.at[dst_v.at[bb]], d_rows.at[bb], sem.at[bb])
            pltpu.async_copy(wh_hbm.at[idx2_v.at[bb]], wh_v.at[bb], sem.at[bb])

        fire(0, start)

        def chunk(i, carry):
            b = jnp.bitwise_and(i, 1)
            nb = 1 - b

            @pl.when(i + 1 < nk)
            def _():
                fire(nb, start + i + 1)

            pltpu.make_async_copy(
                s_hbm.at[pl.ds(0, _C)], s_rows.at[b], sem.at[b]).wait()
            pltpu.make_async_copy(
                d_hbm.at[pl.ds(0, _C)], d_rows.at[b], sem.at[b]).wait()
            pltpu.make_async_copy(
                wh_hbm.at[pl.ds(0, _C)], wh_v.at[b], sem.at[b]).wait()

            def exrow(r4, carry2):
                for u in range(4):
                    r = r4 * 4 + u
                    e = s_rows[b, r, :] + d_rows[b, r, :]
                    lr = jnp.maximum(e, 0.2 * e)
                    ex_v[r, :] = jnp.exp(lr - mvec)
                return carry2
            lax.fori_loop(0, _C // 4, exrow, 0)

            def bsel(exv, g):
                vs = []
                for ls in (lanes0, lanes1):
                    h = ls[g]
                    vs.append(lax.squeeze(
                        lax.slice_in_dim(exv, h, h + 1, axis=0),
                        dimensions=[0]))
                return jnp.where(c == 0, vs[0], vs[1])

            def wrow(r2, carry2):
                for u in range(2):
                    r = r2 * 2 + u
                    row16 = ex_v[r, :]
                    wv = {}
                    for g in range(groups):
                        h = (lanes0[g], lanes1[g])
                        if h not in wv:
                            wv[h] = bsel(row16, g)
                        wh_v[b, r, pl.ds(g * 16, 16)] = (
                            wh_v[b, r, pl.ds(g * 16, 16)] * wv[h])
                return carry2
            lax.fori_loop(0, _C // 2, wrow, 0)

            pltpu.sync_copy(wh_v.at[b], num_sh.at[dst_v.at[b]], add=True)
            pltpu.sync_copy(ex_v, den_sh.at[idxd_v.at[b]], add=True)
            return carry
        lax.fori_loop(0, nk, chunk, 0)

        plsc.subcore_barrier()
        pltpu.sync_copy(num_sh.at[pl.ds(r0, _RPT)],
                        num_out.at[c, pl.ds(r0, _RPT)])
        pltpu.sync_copy(den_sh.at[pl.ds(rd0, _RPTD)],
                        den_out.at[pl.ds(c * _NH + rd0, _RPTD)])

    return edge_kernel


_edge1 = _make_edge_kernel(48, 16, 0)
_edge2 = _make_edge_kernel(96, 32, 0)
_edge3a = _make_edge_kernel(96, 64, 0)
_edge3b = _make_edge_kernel(96, 64, 1)


def _elu(x):
    return jnp.where(x > 0, x, jnp.exp(jnp.minimum(x, 0.0)) - 1.0)


def _dense_common(i, h, w_ref, as_ref, ad_ref, wh_ref, s_ref, d_ref, m_ref,
                  nparts):
    wh = jnp.dot(h, w_ref[...], preferred_element_type=jnp.float32)
    s16 = jnp.dot(wh, as_ref[...], preferred_element_type=jnp.float32)
    d16 = jnp.dot(wh, ad_ref[...], preferred_element_type=jnp.float32)
    part = wh.shape[1] // nparts
    for j in range(nparts):
        wh_ref[j] = wh[:, j * part:(j + 1) * part]
    s_ref[...] = s16
    d_ref[...] = d16
    cur = jnp.concatenate(
        [jnp.max(s16, axis=0, keepdims=True),
         jnp.max(d16, axis=0, keepdims=True)], axis=0)

    @pl.when(i == 0)
    def _():
        m_ref[...] = cur

    @pl.when(i > 0)
    def _():
        m_ref[...] = jnp.maximum(m_ref[...], cur)


def _dense_outs(hfn, nparts):
    part = hfn // nparts
    out_specs = [
        pl.BlockSpec((nparts, _BN, part), lambda i: (0, i, 0)),
        pl.BlockSpec((_BN, 16), lambda i: (i, 0)),
        pl.BlockSpec((_BN, 16), lambda i: (i, 0)),
        pl.BlockSpec((2, 16), lambda i: (0, 0)),
    ]
    out_shape = [
        jax.ShapeDtypeStruct((nparts, _N, part), jnp.float32),
        jax.ShapeDtypeStruct((_N, 16), jnp.float32),
        jax.ShapeDtypeStruct((_N, 16), jnp.float32),
        jax.ShapeDtypeStruct((2, 16), jnp.float32),
    ]
    return out_specs, out_shape


def _make_dense1(hfn, nparts):
    def body(x_ref, w_ref, as_ref, ad_ref, wh_ref, s_ref, d_ref, m_ref):
        i = pl.program_id(0)
        _dense_common(i, x_ref[...], w_ref, as_ref, ad_ref,
                      wh_ref, s_ref, d_ref, m_ref, nparts)

    din = 11
    out_specs, out_shape = _dense_outs(hfn, nparts)
    return pl.pallas_call(
        body,
        grid=(_NBLK,),
        in_specs=[
            pl.BlockSpec((_BN, din), lambda i: (i, 0)),
            pl.BlockSpec((din, hfn), lambda i: (0, 0)),
            pl.BlockSpec((hfn, 16), lambda i: (0, 0)),
            pl.BlockSpec((hfn, 16), lambda i: (0, 0)),
        ],
        out_specs=out_specs,
        out_shape=out_shape,
    )


def _finalize(num_refs, den_ref, e_ref):
    cat = jnp.concatenate([r[j] for r in num_refs for j in (0, 1)], axis=1)
    dexp = jnp.dot(den_ref[...], e_ref[...], preferred_element_type=jnp.float32)
    return _elu(cat / (dexp + 1e-16))


def _make_dense(hfp, hfn, nin, nparts):
    part_p = hfp // (2 * nin)

    def body(*refs):
        num_refs = refs[:nin]
        den_ref, e_ref, w_ref, as_ref, ad_ref = refs[nin:nin + 5]
        wh_ref, s_ref, d_ref, m_ref = refs[nin + 5:]
        i = pl.program_id(0)
        h = _finalize(num_refs, den_ref, e_ref)
        _dense_common(i, h, w_ref, as_ref, ad_ref,
                      wh_ref, s_ref, d_ref, m_ref, nparts)

    out_specs, out_shape = _dense_outs(hfn, nparts)
    return pl.pallas_call(
        body,
        grid=(_NBLK,),
        in_specs=[
            pl.BlockSpec((2, _BN, part_p), lambda i: (0, i, 0))
            for _ in range(nin)
        ] + [
            pl.BlockSpec((_BN, 16), lambda i: (i, 0)),
            pl.BlockSpec((16, hfp), lambda i: (0, 0)),
            pl.BlockSpec((hfp, hfn), lambda i: (0, 0)),
            pl.BlockSpec((hfn, 16), lambda i: (0, 0)),
            pl.BlockSpec((hfn, 16), lambda i: (0, 0)),
        ],
        out_specs=out_specs,
        out_shape=out_shape,
    )


def _make_readout(hfp, nin):
    part_p = hfp // (2 * nin)

    def body(*refs):
        ni_refs = refs[:nin]
        di_ref = refs[nin]
        nn_refs = refs[nin + 1:2 * nin + 1]
        dn_ref, e_ref, wd_ref, bd_ref = refs[2 * nin + 1:2 * nin + 5]
        acc_ref, out_ref = refs[2 * nin + 5:]
        i = pl.program_id(0)
        hi = _finalize(ni_refs, di_ref, e_ref)
        hn = _finalize(nn_refs, dn_ref, e_ref)
        both = jnp.concatenate([hi, hn], axis=1)
        ps = jnp.sum(both, axis=0, keepdims=True)

        @pl.when(i == 0)
        def _():
            acc_ref[...] = ps

        @pl.when(i > 0)
        def _():
            acc_ref[...] = acc_ref[...] + ps

        @pl.when(i == _NBLK - 1)
        def _():
            s = acc_ref[...]
            nrm = jnp.maximum(jnp.sqrt(jnp.sum(s * s)), 1e-12)
            out_ref[...] = (jnp.dot(s / nrm, wd_ref[...],
                                    preferred_element_type=jnp.float32)
                            + bd_ref[...])

    num_spec = pl.BlockSpec((2, _BN, part_p), lambda i: (0, i, 0))
    den_spec = pl.BlockSpec((_BN, 16), lambda i: (i, 0))
    return pl.pallas_call(
        body,
        grid=(_NBLK,),
        in_specs=[num_spec] * nin + [den_spec] + [num_spec] * nin + [
            den_spec,
            pl.BlockSpec((16, hfp), lambda i: (0, 0)),
            pl.BlockSpec((2 * hfp, 1), lambda i: (0, 0)),
            pl.BlockSpec((1, 1), lambda i: (0, 0)),
        ],
        out_specs=[
            pl.BlockSpec((1, 2 * hfp), lambda i: (0, 0)),
            pl.BlockSpec((1, 1), lambda i: (0, 0)),
        ],
        out_shape=[
            jax.ShapeDtypeStruct((1, 2 * hfp), jnp.float32),
            jax.ShapeDtypeStruct((1, 1), jnp.float32),
        ],
    )


_dense1 = _make_dense1(96, 2)
_dense2 = _make_dense(96, 192, 1, 2)
_dense3 = _make_dense(192, 384, 1, 4)
_readout = _make_readout(384, 2)


def _blockdiag16(a):
    h, f = a.shape
    rows = jnp.arange(h * f)
    heads = rows // f
    flat = a.reshape(-1)
    out = jnp.zeros((h * f, 16), jnp.float32)
    out = out.at[rows, heads].set(flat)
    out = out.at[rows, heads + 8].set(flat)
    return out


def _expand16(hf, f):
    cols = jnp.arange(hf)
    heads = cols // f
    return jnp.zeros((16, hf), jnp.float32).at[heads, cols].set(1.0)


def kernel(x, edge_index_int, edge_index_nh, W1, a1_src, a1_dst,
           W2, a2_src, a2_dst, W3, a3_src, a3_dst, Wd, bd):
    a1s, a1d = _blockdiag16(a1_src), _blockdiag16(a1_dst)
    a2s, a2d = _blockdiag16(a2_src), _blockdiag16(a2_dst)
    a3s, a3d = _blockdiag16(a3_src), _blockdiag16(a3_dst)
    e1 = _expand16(96, 16)
    e2 = _expand16(192, 32)
    e3 = _expand16(384, 64)

    si, di_ = edge_index_int[0], edge_index_int[1]
    sn, dn_ = edge_index_nh[0], edge_index_nh[1]

    wh1, s1, d1, m1 = _dense1(x, W1, a1s, a1d)
    wh1f = wh1.reshape(2 * _N, 48)
    ni1, den_i1 = _edge1(si, di_, wh1f, s1, d1, m1)
    nn1, den_n1 = _edge1(sn, dn_, wh1f, s1, d1, m1)

    wh2i, s2i, d2i, m2i = _dense2(ni1, den_i1, e1, W2, a2s, a2d)
    wh2n, s2n, d2n, m2n = _dense2(nn1, den_n1, e1, W2, a2s, a2d)
    ni2, den_i2 = _edge2(si, di_, wh2i.reshape(2 * _N, 96), s2i, d2i, m2i)
    nn2, den_n2 = _edge2(sn, dn_, wh2n.reshape(2 * _N, 96), s2n, d2n, m2n)

    wh3i, s3i, d3i, m3i = _dense3(ni2, den_i2, e2, W3, a3s, a3d)
    wh3n, s3n, d3n, m3n = _dense3(nn2, den_n2, e2, W3, a3s, a3d)
    wh3i4 = wh3i.reshape(4 * _N, 96)
    wh3n4 = wh3n.reshape(4 * _N, 96)
    ni3a, den_i3 = _edge3a(si, di_, wh3i4[:2 * _N], s3i, d3i, m3i)
    ni3b, _ = _edge3b(si, di_, wh3i4[2 * _N:], s3i, d3i, m3i)
    nn3a, den_n3 = _edge3a(sn, dn_, wh3n4[:2 * _N], s3n, d3n, m3n)
    nn3b, _ = _edge3b(sn, dn_, wh3n4[2 * _N:], s3n, d3n, m3n)

    _, out = _readout(ni3a, ni3b, den_i3, nn3a, nn3b, den_n3,
                      e3, Wd, bd.reshape(1, 1))
    return out.reshape((1,))


# speedup vs baseline: 1.3245x; 1.3245x over previous
import functools

import jax
import jax.numpy as jnp
from jax import lax
from jax.experimental import pallas as pl
from jax.experimental.pallas import tpu as pltpu
from jax.experimental.pallas import tpu_sc as plsc

_N = 10000
_E = 160000
_HEADS = 6
_NCORES = 2
_NSUB = 16
_NW = _NCORES * _NSUB
_C = 128
_NCHUNKS = _E // _C
_CHUNK_BASE = _NCHUNKS // _NW
_CHUNK_REM = _NCHUNKS - _CHUNK_BASE * _NW
_NPAD = 10240
_RPT = _NPAD // _NSUB
_ZR = 128
_NH = _NPAD // 2
_NHT = _NH + _ZR
_RPTD = _NH // _NSUB
_BN = 1000
_NBLK = _N // _BN


def _make_edge_kernel(hf2, filt, q):
    groups = hf2 // 16
    gpf = filt // 16
    lanes0 = [((2 * q + 0) * hf2 // 16 + g) // gpf for g in range(groups)]
    lanes1 = [((2 * q + 1) * hf2 // 16 + g) // gpf for g in range(groups)]
    mesh = plsc.VectorSubcoreMesh(core_axis_name="c", subcore_axis_name="s",
                                  num_cores=_NCORES, num_subcores=_NSUB)

    out_type = (
        jax.ShapeDtypeStruct((_NCORES, _NPAD, hf2), jnp.float32),
        jax.ShapeDtypeStruct((_NPAD, 16), jnp.float32),
    )
    scratch_types = [
        pltpu.VMEM((_C,), jnp.int32),
        pltpu.VMEM((_C,), jnp.int32),
        pltpu.VMEM((_C,), jnp.int32),
        pltpu.VMEM((_C,), jnp.int32),
        pltpu.VMEM((_C, 16), jnp.float32),
        pltpu.VMEM((_C, 16), jnp.float32),
        pltpu.VMEM((_C, 16), jnp.float32),
        pltpu.VMEM((_C, hf2), jnp.float32),
        pltpu.VMEM((2, 16), jnp.float32),
        pltpu.VMEM((_ZR, hf2), jnp.float32),
        pltpu.VMEM((_ZR, 16), jnp.float32),
        pltpu.VMEM_SHARED((_NPAD, hf2), jnp.float32),
        pltpu.VMEM_SHARED((_NHT, 16), jnp.float32),
        pltpu.SemaphoreType.DMA,
    ]

    @functools.partial(
        pl.kernel, mesh=mesh, out_type=out_type, scratch_types=scratch_types,
        compiler_params=pltpu.CompilerParams(needs_layout_passes=False,
                                             use_tc_tiling_on_sc=False))
    def edge_kernel(src_hbm, dst_hbm, wh_hbm, s_hbm, d_hbm, m_hbm,
                    num_out, den_out,
                    src_v, dst_v, idx2_v, idxd_v, s_rows, d_rows, ex_v, wh_v,
                    m_v, zb, zbd, num_sh, den_sh, sem):
        c = lax.axis_index("c")
        sid = lax.axis_index("s")
        zvec = jnp.zeros((16,), jnp.float32)

        def zrow(r, carry):
            for g in range(groups):
                zb[r, pl.ds(g * 16, 16)] = zvec
            zbd[r, :] = zvec
            return carry
        lax.fori_loop(0, _ZR, zrow, 0)

        r0 = sid * _RPT
        for k in range(_RPT // _ZR):
            pltpu.sync_copy(zb, num_sh.at[pl.ds(r0 + k * _ZR, _ZR)])
        rd0 = sid * _RPTD
        for k in range(_RPTD // _ZR):
            pltpu.sync_copy(zbd, den_sh.at[pl.ds(rd0 + k * _ZR, _ZR)])
        pltpu.sync_copy(zbd.at[pl.ds(0, 8)], den_sh.at[pl.ds(_NH + sid * 8, 8)])
        pltpu.sync_copy(m_hbm, m_v)
        plsc.subcore_barrier()

        msum = m_v[0, :] + m_v[1, :]
        mvec = jnp.maximum(msum, 0.2 * msum)
        lanes = jax.lax.iota(jnp.int32, 16)
        trash = lanes % 8 + (_NH + sid * 8)
        coff = c * _N
        dlo = c * _NH

        w = sid * _NCORES + c
        start = w * _CHUNK_BASE + jnp.minimum(w, _CHUNK_REM)
        nk = _CHUNK_BASE + jnp.where(w < _CHUNK_REM, 1, 0)

        def wloop(glanes):
            def wrow(r2, carry2):
                for u in range(2):
                    r = r2 * 2 + u
                    row16 = ex_v[r, :]
                    wv = {}
                    for g in range(groups):
                        h = glanes[g]
                        if h not in wv:
                            wv[h] = lax.squeeze(
                                lax.slice_in_dim(row16, h, h + 1, axis=0),
                                dimensions=[0])
                        wh_v[r, pl.ds(g * 16, 16)] = (
                            wh_v[r, pl.ds(g * 16, 16)] * wv[h])
                return carry2
            lax.fori_loop(0, _C // 2, wrow, 0)

        def chunk(i, carry):
            base = (start + i) * _C
            pltpu.sync_copy(src_hbm.at[pl.ds(base, _C)], src_v)
            pltpu.sync_copy(dst_hbm.at[pl.ds(base, _C)], dst_v)

            def addl(j, carry2):
                sl = pl.ds(j * 16, 16)
                idx2_v[sl] = src_v[sl] + coff
                t = dst_v[sl] - dlo
                oob = (t < 0) | (t >= _NH)
                idxd_v[sl] = jnp.where(oob, trash, t)
                return carry2
            lax.fori_loop(0, _C // 16, addl, 0)

            cp1 = pltpu.async_copy(s_hbm.at[src_v], s_rows, sem)
            cp2 = pltpu.async_copy(d_hbm.at[dst_v], d_rows, sem)
            cp3 = pltpu.async_copy(wh_hbm.at[idx2_v], wh_v, sem)
            cp1.wait()
            cp2.wait()
            cp3.wait()

            def exrow(r4, carry2):
                rs = [r4 * 4 + u for u in range(4)]
                svs = [s_rows[r, :] for r in rs]
                dvs = [d_rows[r, :] for r in rs]
                es = [s + d for s, d in zip(svs, dvs)]
                lrs = [jnp.maximum(e, 0.2 * e) for e in es]
                exs = [jnp.exp(lr - mvec) for lr in lrs]
                for r, ex in zip(rs, exs):
                    ex_v[r, :] = ex
                return carry2
            lax.fori_loop(0, _C // 4, exrow, 0)

            @pl.when(c == 0)
            def _():
                wloop(lanes0)

            @pl.when(c == 1)
            def _():
                wloop(lanes1)

            pltpu.sync_copy(wh_v, num_sh.at[dst_v], add=True)
            pltpu.sync_copy(ex_v, den_sh.at[idxd_v], add=True)
            return carry
        lax.fori_loop(0, nk, chunk, 0)

        plsc.subcore_barrier()
        pltpu.sync_copy(num_sh.at[pl.ds(r0, _RPT)],
                        num_out.at[c, pl.ds(r0, _RPT)])
        pltpu.sync_copy(den_sh.at[pl.ds(rd0, _RPTD)],
                        den_out.at[pl.ds(c * _NH + rd0, _RPTD)])

    return edge_kernel


_edge1 = _make_edge_kernel(48, 16, 0)
_edge2 = _make_edge_kernel(96, 32, 0)
_edge3a = _make_edge_kernel(96, 64, 0)
_edge3b = _make_edge_kernel(96, 64, 1)


def _elu(x):
    return jnp.where(x > 0, x, jnp.exp(jnp.minimum(x, 0.0)) - 1.0)


def _dense_common(i, h, w_ref, as_ref, ad_ref, wh_ref, s_ref, d_ref, m_ref,
                  nparts):
    wh = jnp.dot(h, w_ref[...], preferred_element_type=jnp.float32)
    s16 = jnp.dot(wh, as_ref[...], preferred_element_type=jnp.float32)
    d16 = jnp.dot(wh, ad_ref[...], preferred_element_type=jnp.float32)
    part = wh.shape[1] // nparts
    for j in range(nparts):
        wh_ref[j] = wh[:, j * part:(j + 1) * part]
    s_ref[...] = s16
    d_ref[...] = d16
    cur = jnp.concatenate(
        [jnp.max(s16, axis=0, keepdims=True),
         jnp.max(d16, axis=0, keepdims=True)], axis=0)

    @pl.when(i == 0)
    def _():
        m_ref[...] = cur

    @pl.when(i > 0)
    def _():
        m_ref[...] = jnp.maximum(m_ref[...], cur)


def _dense_outs(hfn, nparts):
    part = hfn // nparts
    out_specs = [
        pl.BlockSpec((nparts, _BN, part), lambda i: (0, i, 0)),
        pl.BlockSpec((_BN, 16), lambda i: (i, 0)),
        pl.BlockSpec((_BN, 16), lambda i: (i, 0)),
        pl.BlockSpec((2, 16), lambda i: (0, 0)),
    ]
    out_shape = [
        jax.ShapeDtypeStruct((nparts, _N, part), jnp.float32),
        jax.ShapeDtypeStruct((_N, 16), jnp.float32),
        jax.ShapeDtypeStruct((_N, 16), jnp.float32),
        jax.ShapeDtypeStruct((2, 16), jnp.float32),
    ]
    return out_specs, out_shape


def _make_dense1(hfn, nparts):
    def body(x_ref, w_ref, as_ref, ad_ref, wh_ref, s_ref, d_ref, m_ref):
        i = pl.program_id(0)
        _dense_common(i, x_ref[...], w_ref, as_ref, ad_ref,
                      wh_ref, s_ref, d_ref, m_ref, nparts)

    din = 11
    out_specs, out_shape = _dense_outs(hfn, nparts)
    return pl.pallas_call(
        body,
        grid=(_NBLK,),
        in_specs=[
            pl.BlockSpec((_BN, din), lambda i: (i, 0)),
            pl.BlockSpec((din, hfn), lambda i: (0, 0)),
            pl.BlockSpec((hfn, 16), lambda i: (0, 0)),
            pl.BlockSpec((hfn, 16), lambda i: (0, 0)),
        ],
        out_specs=out_specs,
        out_shape=out_shape,
    )


def _finalize(num_refs, den_ref, e_ref):
    cat = jnp.concatenate([r[j] for r in num_refs for j in (0, 1)], axis=1)
    dexp = jnp.dot(den_ref[...], e_ref[...], preferred_element_type=jnp.float32)
    return _elu(cat / (dexp + 1e-16))


def _make_dense(hfp, hfn, nin, nparts):
    part_p = hfp // (2 * nin)

    def body(*refs):
        num_refs = refs[:nin]
        den_ref, e_ref, w_ref, as_ref, ad_ref = refs[nin:nin + 5]
        wh_ref, s_ref, d_ref, m_ref = refs[nin + 5:]
        i = pl.program_id(0)
        h = _finalize(num_refs, den_ref, e_ref)
        _dense_common(i, h, w_ref, as_ref, ad_ref,
                      wh_ref, s_ref, d_ref, m_ref, nparts)

    out_specs, out_shape = _dense_outs(hfn, nparts)
    return pl.pallas_call(
        body,
        grid=(_NBLK,),
        in_specs=[
            pl.BlockSpec((2, _BN, part_p), lambda i: (0, i, 0))
            for _ in range(nin)
        ] + [
            pl.BlockSpec((_BN, 16), lambda i: (i, 0)),
            pl.BlockSpec((16, hfp), lambda i: (0, 0)),
            pl.BlockSpec((hfp, hfn), lambda i: (0, 0)),
            pl.BlockSpec((hfn, 16), lambda i: (0, 0)),
            pl.BlockSpec((hfn, 16), lambda i: (0, 0)),
        ],
        out_specs=out_specs,
        out_shape=out_shape,
    )


def _make_readout(hfp, nin):
    part_p = hfp // (2 * nin)

    def body(*refs):
        ni_refs = refs[:nin]
        di_ref = refs[nin]
        nn_refs = refs[nin + 1:2 * nin + 1]
        dn_ref, e_ref, wd_ref, bd_ref = refs[2 * nin + 1:2 * nin + 5]
        acc_ref, out_ref = refs[2 * nin + 5:]
        i = pl.program_id(0)
        hi = _finalize(ni_refs, di_ref, e_ref)
        hn = _finalize(nn_refs, dn_ref, e_ref)
        both = jnp.concatenate([hi, hn], axis=1)
        ps = jnp.sum(both, axis=0, keepdims=True)

        @pl.when(i == 0)
        def _():
            acc_ref[...] = ps

        @pl.when(i > 0)
        def _():
            acc_ref[...] = acc_ref[...] + ps

        @pl.when(i == _NBLK - 1)
        def _():
            s = acc_ref[...]
            nrm = jnp.maximum(jnp.sqrt(jnp.sum(s * s)), 1e-12)
            out_ref[...] = (jnp.dot(s / nrm, wd_ref[...],
                                    preferred_element_type=jnp.float32)
                            + bd_ref[...])

    num_spec = pl.BlockSpec((2, _BN, part_p), lambda i: (0, i, 0))
    den_spec = pl.BlockSpec((_BN, 16), lambda i: (i, 0))
    return pl.pallas_call(
        body,
        grid=(_NBLK,),
        in_specs=[num_spec] * nin + [den_spec] + [num_spec] * nin + [
            den_spec,
            pl.BlockSpec((16, hfp), lambda i: (0, 0)),
            pl.BlockSpec((2 * hfp, 1), lambda i: (0, 0)),
            pl.BlockSpec((1, 1), lambda i: (0, 0)),
        ],
        out_specs=[
            pl.BlockSpec((1, 2 * hfp), lambda i: (0, 0)),
            pl.BlockSpec((1, 1), lambda i: (0, 0)),
        ],
        out_shape=[
            jax.ShapeDtypeStruct((1, 2 * hfp), jnp.float32),
            jax.ShapeDtypeStruct((1, 1), jnp.float32),
        ],
    )


_dense1 = _make_dense1(96, 2)
_dense2 = _make_dense(96, 192, 1, 2)
_dense3 = _make_dense(192, 384, 1, 4)
_readout = _make_readout(384, 2)


def _blockdiag16(a):
    h, f = a.shape
    rows = jnp.arange(h * f)
    heads = rows // f
    flat = a.reshape(-1)
    out = jnp.zeros((h * f, 16), jnp.float32)
    out = out.at[rows, heads].set(flat)
    out = out.at[rows, heads + 8].set(flat)
    return out


def _expand16(hf, f):
    cols = jnp.arange(hf)
    heads = cols // f
    return jnp.zeros((16, hf), jnp.float32).at[heads, cols].set(1.0)


def kernel(x, edge_index_int, edge_index_nh, W1, a1_src, a1_dst,
           W2, a2_src, a2_dst, W3, a3_src, a3_dst, Wd, bd):
    a1s, a1d = _blockdiag16(a1_src), _blockdiag16(a1_dst)
    a2s, a2d = _blockdiag16(a2_src), _blockdiag16(a2_dst)
    a3s, a3d = _blockdiag16(a3_src), _blockdiag16(a3_dst)
    e1 = _expand16(96, 16)
    e2 = _expand16(192, 32)
    e3 = _expand16(384, 64)

    si, di_ = edge_index_int[0], edge_index_int[1]
    sn, dn_ = edge_index_nh[0], edge_index_nh[1]

    wh1, s1, d1, m1 = _dense1(x, W1, a1s, a1d)
    wh1f = wh1.reshape(2 * _N, 48)
    ni1, den_i1 = _edge1(si, di_, wh1f, s1, d1, m1)
    nn1, den_n1 = _edge1(sn, dn_, wh1f, s1, d1, m1)

    wh2i, s2i, d2i, m2i = _dense2(ni1, den_i1, e1, W2, a2s, a2d)
    wh2n, s2n, d2n, m2n = _dense2(nn1, den_n1, e1, W2, a2s, a2d)
    ni2, den_i2 = _edge2(si, di_, wh2i.reshape(2 * _N, 96), s2i, d2i, m2i)
    nn2, den_n2 = _edge2(sn, dn_, wh2n.reshape(2 * _N, 96), s2n, d2n, m2n)

    wh3i, s3i, d3i, m3i = _dense3(ni2, den_i2, e2, W3, a3s, a3d)
    wh3n, s3n, d3n, m3n = _dense3(nn2, den_n2, e2, W3, a3s, a3d)
    wh3i4 = wh3i.reshape(4 * _N, 96)
    wh3n4 = wh3n.reshape(4 * _N, 96)
    ni3a, den_i3 = _edge3a(si, di_, wh3i4[:2 * _N], s3i, d3i, m3i)
    ni3b, _ = _edge3b(si, di_, wh3i4[2 * _N:], s3i, d3i, m3i)
    nn3a, den_n3 = _edge3a(sn, dn_, wh3n4[:2 * _N], s3n, d3n, m3n)
    nn3b, _ = _edge3b(sn, dn_, wh3n4[2 * _N:], s3n, d3n, m3n)

    _, out = _readout(ni3a, ni3b, den_i3, nn3a, nn3b, den_n3,
                      e3, Wd, bd.reshape(1, 1))
    return out.reshape((1,))


# speedup vs baseline: 2.1799x; 1.6458x over previous
import functools

import jax
import jax.numpy as jnp
from jax import lax
from jax.experimental import pallas as pl
from jax.experimental.pallas import tpu as pltpu
from jax.experimental.pallas import tpu_sc as plsc

_N = 10000
_E = 160000
_HEADS = 6
_NCORES = 2
_NSUB = 16
_NW = _NCORES * _NSUB
_C = 128
_TPW = 40
_PAIRS = _TPW // 2
_NCHP = _NW * _TPW
_EP = _NCHP * _C
_NPAD = 10240
_RPT = _NPAD // _NSUB
_ZR = 32
_NH = _NPAD // 2
_NHT = _NH + 8 * _NSUB
_RPTD = _NH // _NSUB
_BN = 1000
_NBLK = _N // _BN


def _make_edge_kernel(hf2, filt, q):
    groups = hf2 // 16
    gpf = filt // 16
    lanes0 = [((2 * q + 0) * hf2 // 16 + g) // gpf for g in range(groups)]
    lanes1 = [((2 * q + 1) * hf2 // 16 + g) // gpf for g in range(groups)]
    mesh = plsc.VectorSubcoreMesh(core_axis_name="c", subcore_axis_name="s",
                                  num_cores=_NCORES, num_subcores=_NSUB)

    out_type = (
        jax.ShapeDtypeStruct((_NCORES, _NPAD, hf2), jnp.float32),
        jax.ShapeDtypeStruct((_NPAD, 16), jnp.float32),
    )
    scratch_types = [
        pltpu.VMEM((_TPW, _C), jnp.int32),
        pltpu.VMEM((_TPW, _C), jnp.int32),
        pltpu.VMEM((_TPW, _C), jnp.int32),
        pltpu.VMEM((_TPW, _C), jnp.int32),
        pltpu.VMEM((_C, 16), jnp.float32),
        pltpu.VMEM((_C, 16), jnp.float32),
        pltpu.VMEM((_C, 16), jnp.float32),
        pltpu.VMEM((_C, 16), jnp.float32),
        pltpu.VMEM((_C, 16), jnp.float32),
        pltpu.VMEM((_C, 16), jnp.float32),
        pltpu.VMEM((_C, hf2), jnp.float32),
        pltpu.VMEM((_C, hf2), jnp.float32),
        pltpu.VMEM((2, 16), jnp.float32),
        pltpu.VMEM((_ZR, hf2), jnp.float32),
        pltpu.VMEM((_ZR, 16), jnp.float32),
        pltpu.VMEM_SHARED((_NPAD, hf2), jnp.float32),
        pltpu.VMEM_SHARED((_NHT, 16), jnp.float32),
        pltpu.SemaphoreType.DMA,
        pltpu.SemaphoreType.DMA,
        pltpu.SemaphoreType.DMA,
        pltpu.SemaphoreType.DMA,
    ]

    @functools.partial(
        pl.kernel, mesh=mesh, out_type=out_type, scratch_types=scratch_types,
        compiler_params=pltpu.CompilerParams(needs_layout_passes=False,
                                             use_tc_tiling_on_sc=False))
    def edge_kernel(src_hbm, dst_hbm, wh_hbm, s_hbm, d_hbm, m_hbm,
                    num_out, den_out,
                    src_all, dst_all, idx2_all, idxd_all,
                    s_a, s_b, d_a, d_b, ex_a, ex_b, wh_a, wh_b,
                    m_v, zb, zbd, num_sh, den_sh, gs_a, gs_b, ss_a, ss_b):
        c = lax.axis_index("c")
        sid = lax.axis_index("s")
        zvec = jnp.zeros((16,), jnp.float32)

        def zrow(r, carry):
            for g in range(groups):
                zb[r, pl.ds(g * 16, 16)] = zvec
            zbd[r, :] = zvec
            return carry
        lax.fori_loop(0, _ZR, zrow, 0)

        r0 = sid * _RPT
        for k in range(_RPT // _ZR):
            pltpu.sync_copy(zb, num_sh.at[pl.ds(r0 + k * _ZR, _ZR)])
        rd0 = sid * _RPTD
        for k in range(_RPTD // _ZR):
            pltpu.sync_copy(zbd, den_sh.at[pl.ds(rd0 + k * _ZR, _ZR)])
        pltpu.sync_copy(zbd.at[pl.ds(0, 8)], den_sh.at[pl.ds(_NH + sid * 8, 8)])
        pltpu.sync_copy(m_hbm, m_v)

        w = sid * _NCORES + c
        start = w * _TPW
        pltpu.sync_copy(src_hbm.at[pl.ds(start, _TPW)], src_all)
        pltpu.sync_copy(dst_hbm.at[pl.ds(start, _TPW)], dst_all)
        plsc.subcore_barrier()

        msum = m_v[0, :] + m_v[1, :]
        mvec = jnp.maximum(msum, 0.2 * msum)
        lanes = jax.lax.iota(jnp.int32, 16)
        trash = lanes % 8 + (_NH + sid * 8)
        coff = c * _N
        dlo = c * _NH

        def prep(row, carry):
            for j in range(_C // 16):
                sl = pl.ds(j * 16, 16)
                idx2_all[row, sl] = src_all[row, sl] + coff
                t = dst_all[row, sl] - dlo
                oob = (t < 0) | (t >= _NH)
                idxd_all[row, sl] = jnp.where(oob, trash, t)
            return carry
        lax.fori_loop(0, _TPW, prep, 0)

        def fire_g(sv, dv, whv, gsem, k):
            pltpu.async_copy(s_hbm.at[src_all.at[k]], sv, gsem)
            pltpu.async_copy(d_hbm.at[dst_all.at[k]], dv, gsem)
            pltpu.async_copy(wh_hbm.at[idx2_all.at[k]], whv, gsem)

        def drain_g(sv, dv, whv, gsem):
            pltpu.make_async_copy(s_hbm.at[pl.ds(0, _C)], sv, gsem).wait()
            pltpu.make_async_copy(d_hbm.at[pl.ds(0, _C)], dv, gsem).wait()
            pltpu.make_async_copy(wh_hbm.at[pl.ds(0, _C)], whv, gsem).wait()

        def fire_s(whv, exv, ssem, k):
            pltpu.async_copy(whv, num_sh.at[dst_all.at[k]], ssem, add=True)
            pltpu.async_copy(exv, den_sh.at[idxd_all.at[k]], ssem, add=True)

        def drain_s(ssem):
            pltpu.make_async_copy(
                wh_hbm.at[pl.ds(0, _C)], num_sh.at[pl.ds(0, _C)], ssem).wait()
            pltpu.make_async_copy(
                s_hbm.at[pl.ds(0, _C)], den_sh.at[pl.ds(0, _C)], ssem).wait()

        def compute(s_rows, d_rows, ex_v, wh_v):
            def exrow(r4, carry2):
                rs = [r4 * 4 + u for u in range(4)]
                svs = [s_rows[r, :] for r in rs]
                dvs = [d_rows[r, :] for r in rs]
                es = [s + d for s, d in zip(svs, dvs)]
                lrs = [jnp.maximum(e, 0.2 * e) for e in es]
                exs = [jnp.exp(lr - mvec) for lr in lrs]
                for r, ex in zip(rs, exs):
                    ex_v[r, :] = ex
                return carry2
            lax.fori_loop(0, _C // 4, exrow, 0)

            def wloop(glanes):
                def wrow(r2, carry2):
                    for u in range(2):
                        r = r2 * 2 + u
                        row16 = ex_v[r, :]
                        wv = {}
                        for g in range(groups):
                            h = glanes[g]
                            if h not in wv:
                                wv[h] = lax.squeeze(
                                    lax.slice_in_dim(row16, h, h + 1, axis=0),
                                    dimensions=[0])
                            wh_v[r, pl.ds(g * 16, 16)] = (
                                wh_v[r, pl.ds(g * 16, 16)] * wv[h])
                    return carry2
                lax.fori_loop(0, _C // 2, wrow, 0)

            @pl.when(c == 0)
            def _():
                wloop(lanes0)

            @pl.when(c == 1)
            def _():
                wloop(lanes1)

        fire_g(s_a, d_a, wh_a, gs_a, 0)

        def pair(j, carry):
            drain_g(s_a, d_a, wh_a, gs_a)

            @pl.when(j > 0)
            def _():
                drain_s(ss_b)
            fire_g(s_b, d_b, wh_b, gs_b, 2 * j + 1)
            compute(s_a, d_a, ex_a, wh_a)
            fire_s(wh_a, ex_a, ss_a, 2 * j)
            drain_g(s_b, d_b, wh_b, gs_b)
            compute(s_b, d_b, ex_b, wh_b)
            drain_s(ss_a)

            @pl.when(j < _PAIRS - 1)
            def _():
                fire_g(s_a, d_a, wh_a, gs_a, 2 * j + 2)
            fire_s(wh_b, ex_b, ss_b, 2 * j + 1)
            return carry
        lax.fori_loop(0, _PAIRS, pair, 0)
        drain_s(ss_b)

        plsc.subcore_barrier()
        pltpu.sync_copy(num_sh.at[pl.ds(r0, _RPT)],
                        num_out.at[c, pl.ds(r0, _RPT)])
        pltpu.sync_copy(den_sh.at[pl.ds(rd0, _RPTD)],
                        den_out.at[pl.ds(c * _NH + rd0, _RPTD)])

    return edge_kernel


_edge1 = _make_edge_kernel(48, 16, 0)
_edge2 = _make_edge_kernel(96, 32, 0)
_edge3a = _make_edge_kernel(96, 64, 0)
_edge3b = _make_edge_kernel(96, 64, 1)


def _elu(x):
    return jnp.where(x > 0, x, jnp.exp(jnp.minimum(x, 0.0)) - 1.0)


def _dense_common(i, h, w_ref, as_ref, ad_ref, wh_ref, s_ref, d_ref, m_ref,
                  nparts):
    wh = jnp.dot(h, w_ref[...], preferred_element_type=jnp.float32)
    s16 = jnp.dot(wh, as_ref[...], preferred_element_type=jnp.float32)
    d16 = jnp.dot(wh, ad_ref[...], preferred_element_type=jnp.float32)
    part = wh.shape[1] // nparts
    for j in range(nparts):
        wh_ref[j] = wh[:, j * part:(j + 1) * part]
    s_ref[...] = s16
    d_ref[...] = d16
    cur = jnp.concatenate(
        [jnp.max(s16, axis=0, keepdims=True),
         jnp.max(d16, axis=0, keepdims=True)], axis=0)

    @pl.when(i == 0)
    def _():
        m_ref[...] = cur

    @pl.when(i > 0)
    def _():
        m_ref[...] = jnp.maximum(m_ref[...], cur)


def _dense_outs(hfn, nparts):
    part = hfn // nparts
    out_specs = [
        pl.BlockSpec((nparts, _BN, part), lambda i: (0, i, 0)),
        pl.BlockSpec((_BN, 16), lambda i: (i, 0)),
        pl.BlockSpec((_BN, 16), lambda i: (i, 0)),
        pl.BlockSpec((2, 16), lambda i: (0, 0)),
    ]
    out_shape = [
        jax.ShapeDtypeStruct((nparts, _N, part), jnp.float32),
        jax.ShapeDtypeStruct((_N, 16), jnp.float32),
        jax.ShapeDtypeStruct((_N, 16), jnp.float32),
        jax.ShapeDtypeStruct((2, 16), jnp.float32),
    ]
    return out_specs, out_shape


def _make_dense1(hfn, nparts):
    def body(x_ref, w_ref, as_ref, ad_ref, wh_ref, s_ref, d_ref, m_ref):
        i = pl.program_id(0)
        _dense_common(i, x_ref[...], w_ref, as_ref, ad_ref,
                      wh_ref, s_ref, d_ref, m_ref, nparts)

    din = 11
    out_specs, out_shape = _dense_outs(hfn, nparts)
    return pl.pallas_call(
        body,
        grid=(_NBLK,),
        in_specs=[
            pl.BlockSpec((_BN, din), lambda i: (i, 0)),
            pl.BlockSpec((din, hfn), lambda i: (0, 0)),
            pl.BlockSpec((hfn, 16), lambda i: (0, 0)),
            pl.BlockSpec((hfn, 16), lambda i: (0, 0)),
        ],
        out_specs=out_specs,
        out_shape=out_shape,
    )


def _finalize(num_refs, den_ref, e_ref):
    cat = jnp.concatenate([r[j] for r in num_refs for j in (0, 1)], axis=1)
    dexp = jnp.dot(den_ref[...], e_ref[...], preferred_element_type=jnp.float32)
    return _elu(cat / (dexp + 1e-16))


def _make_dense(hfp, hfn, nin, nparts):
    part_p = hfp // (2 * nin)

    def body(*refs):
        num_refs = refs[:nin]
        den_ref, e_ref, w_ref, as_ref, ad_ref = refs[nin:nin + 5]
        wh_ref, s_ref, d_ref, m_ref = refs[nin + 5:]
        i = pl.program_id(0)
        h = _finalize(num_refs, den_ref, e_ref)
        _dense_common(i, h, w_ref, as_ref, ad_ref,
                      wh_ref, s_ref, d_ref, m_ref, nparts)

    out_specs, out_shape = _dense_outs(hfn, nparts)
    return pl.pallas_call(
        body,
        grid=(_NBLK,),
        in_specs=[
            pl.BlockSpec((2, _BN, part_p), lambda i: (0, i, 0))
            for _ in range(nin)
        ] + [
            pl.BlockSpec((_BN, 16), lambda i: (i, 0)),
            pl.BlockSpec((16, hfp), lambda i: (0, 0)),
            pl.BlockSpec((hfp, hfn), lambda i: (0, 0)),
            pl.BlockSpec((hfn, 16), lambda i: (0, 0)),
            pl.BlockSpec((hfn, 16), lambda i: (0, 0)),
        ],
        out_specs=out_specs,
        out_shape=out_shape,
    )


def _make_readout(hfp, nin):
    part_p = hfp // (2 * nin)

    def body(*refs):
        ni_refs = refs[:nin]
        di_ref = refs[nin]
        nn_refs = refs[nin + 1:2 * nin + 1]
        dn_ref, e_ref, wd_ref, bd_ref = refs[2 * nin + 1:2 * nin + 5]
        acc_ref, out_ref = refs[2 * nin + 5:]
        i = pl.program_id(0)
        hi = _finalize(ni_refs, di_ref, e_ref)
        hn = _finalize(nn_refs, dn_ref, e_ref)
        both = jnp.concatenate([hi, hn], axis=1)
        ps = jnp.sum(both, axis=0, keepdims=True)

        @pl.when(i == 0)
        def _():
            acc_ref[...] = ps

        @pl.when(i > 0)
        def _():
            acc_ref[...] = acc_ref[...] + ps

        @pl.when(i == _NBLK - 1)
        def _():
            s = acc_ref[...]
            nrm = jnp.maximum(jnp.sqrt(jnp.sum(s * s)), 1e-12)
            out_ref[...] = (jnp.dot(s / nrm, wd_ref[...],
                                    preferred_element_type=jnp.float32)
                            + bd_ref[...])

    num_spec = pl.BlockSpec((2, _BN, part_p), lambda i: (0, i, 0))
    den_spec = pl.BlockSpec((_BN, 16), lambda i: (i, 0))
    return pl.pallas_call(
        body,
        grid=(_NBLK,),
        in_specs=[num_spec] * nin + [den_spec] + [num_spec] * nin + [
            den_spec,
            pl.BlockSpec((16, hfp), lambda i: (0, 0)),
            pl.BlockSpec((2 * hfp, 1), lambda i: (0, 0)),
            pl.BlockSpec((1, 1), lambda i: (0, 0)),
        ],
        out_specs=[
            pl.BlockSpec((1, 2 * hfp), lambda i: (0, 0)),
            pl.BlockSpec((1, 1), lambda i: (0, 0)),
        ],
        out_shape=[
            jax.ShapeDtypeStruct((1, 2 * hfp), jnp.float32),
            jax.ShapeDtypeStruct((1, 1), jnp.float32),
        ],
    )


_dense1 = _make_dense1(96, 2)
_dense2 = _make_dense(96, 192, 1, 2)
_dense3 = _make_dense(192, 384, 1, 4)
_readout = _make_readout(384, 2)


def _blockdiag16(a):
    h, f = a.shape
    rows = jnp.arange(h * f)
    heads = rows // f
    flat = a.reshape(-1)
    out = jnp.zeros((h * f, 16), jnp.float32)
    out = out.at[rows, heads].set(flat)
    out = out.at[rows, heads + 8].set(flat)
    return out


def _expand16(hf, f):
    cols = jnp.arange(hf)
    heads = cols // f
    return jnp.zeros((16, hf), jnp.float32).at[heads, cols].set(1.0)


def kernel(x, edge_index_int, edge_index_nh, W1, a1_src, a1_dst,
           W2, a2_src, a2_dst, W3, a3_src, a3_dst, Wd, bd):
    a1s, a1d = _blockdiag16(a1_src), _blockdiag16(a1_dst)
    a2s, a2d = _blockdiag16(a2_src), _blockdiag16(a2_dst)
    a3s, a3d = _blockdiag16(a3_src), _blockdiag16(a3_dst)
    e1 = _expand16(96, 16)
    e2 = _expand16(192, 32)
    e3 = _expand16(384, 64)

    pad_src = jnp.arange(_EP - _E, dtype=jnp.int32) % _N
    pad_dst = jnp.arange(_EP - _E, dtype=jnp.int32) % (_NPAD - _N) + _N

    def pad2d(sv, dv):
        s2 = jnp.concatenate([sv, pad_src]).reshape(_NCHP, _C)
        d2 = jnp.concatenate([dv, pad_dst]).reshape(_NCHP, _C)
        return s2, d2

    si, di_ = pad2d(edge_index_int[0], edge_index_int[1])
    sn, dn_ = pad2d(edge_index_nh[0], edge_index_nh[1])

    wh1, s1, d1, m1 = _dense1(x, W1, a1s, a1d)
    wh1f = wh1.reshape(2 * _N, 48)
    ni1, den_i1 = _edge1(si, di_, wh1f, s1, d1, m1)
    nn1, den_n1 = _edge1(sn, dn_, wh1f, s1, d1, m1)

    wh2i, s2i, d2i, m2i = _dense2(ni1, den_i1, e1, W2, a2s, a2d)
    wh2n, s2n, d2n, m2n = _dense2(nn1, den_n1, e1, W2, a2s, a2d)
    ni2, den_i2 = _edge2(si, di_, wh2i.reshape(2 * _N, 96), s2i, d2i, m2i)
    nn2, den_n2 = _edge2(sn, dn_, wh2n.reshape(2 * _N, 96), s2n, d2n, m2n)

    wh3i, s3i, d3i, m3i = _dense3(ni2, den_i2, e2, W3, a3s, a3d)
    wh3n, s3n, d3n, m3n = _dense3(nn2, den_n2, e2, W3, a3s, a3d)
    wh3i4 = wh3i.reshape(4 * _N, 96)
    wh3n4 = wh3n.reshape(4 * _N, 96)
    ni3a, den_i3 = _edge3a(si, di_, wh3i4[:2 * _N], s3i, d3i, m3i)
    ni3b, _ = _edge3b(si, di_, wh3i4[2 * _N:], s3i, d3i, m3i)
    nn3a, den_n3 = _edge3a(sn, dn_, wh3n4[:2 * _N], s3n, d3n, m3n)
    nn3b, _ = _edge3b(sn, dn_, wh3n4[2 * _N:], s3n, d3n, m3n)

    _, out = _readout(ni3a, ni3b, den_i3, nn3a, nn3b, den_n3,
                      e3, Wd, bd.reshape(1, 1))
    return out.reshape((1,))


# speedup vs baseline: 2.2598x; 1.0367x over previous
import functools

import jax
import jax.numpy as jnp
from jax import lax
from jax.experimental import pallas as pl
from jax.experimental.pallas import tpu as pltpu
from jax.experimental.pallas import tpu_sc as plsc

_N = 10000
_E = 160000
_HEADS = 6
_NCORES = 2
_NSUB = 16
_NW = _NCORES * _NSUB
_C = 128
_TPW = 40
_PAIRS = _TPW // 2
_NCHP = _NW * _TPW
_EP = _NCHP * _C
_NPAD = 10240
_RPT = _NPAD // _NSUB
_ZR = 32
_NH = _NPAD // 2
_NHT = _NH + 8 * _NSUB
_RPTD = _NH // _NSUB
_BN = 1000
_NBLK = _N // _BN


def _make_edge_kernel(hf2, filt, q):
    groups = hf2 // 16
    gpf = filt // 16
    lanes0 = [((2 * q + 0) * hf2 // 16 + g) // gpf for g in range(groups)]
    lanes1 = [((2 * q + 1) * hf2 // 16 + g) // gpf for g in range(groups)]
    mesh = plsc.VectorSubcoreMesh(core_axis_name="c", subcore_axis_name="s",
                                  num_cores=_NCORES, num_subcores=_NSUB)

    out_type = (
        jax.ShapeDtypeStruct((_NCORES, _NPAD, hf2), jnp.float32),
        jax.ShapeDtypeStruct((_NPAD, 16), jnp.float32),
    )
    scratch_types = [
        pltpu.VMEM((_TPW, _C), jnp.int32),
        pltpu.VMEM((_TPW, _C), jnp.int32),
        pltpu.VMEM((_TPW, _C), jnp.int32),
        pltpu.VMEM((_TPW, _C), jnp.int32),
        pltpu.VMEM((_C, 16), jnp.float32),
        pltpu.VMEM((_C, 16), jnp.float32),
        pltpu.VMEM((_C, 16), jnp.float32),
        pltpu.VMEM((_C, 16), jnp.float32),
        pltpu.VMEM((_C, 16), jnp.float32),
        pltpu.VMEM((_C, 16), jnp.float32),
        pltpu.VMEM((_C, hf2), jnp.float32),
        pltpu.VMEM((_C, hf2), jnp.float32),
        pltpu.VMEM((2, 16), jnp.float32),
        pltpu.VMEM((_ZR, hf2), jnp.float32),
        pltpu.VMEM((_ZR, 16), jnp.float32),
        pltpu.VMEM_SHARED((_NPAD, hf2), jnp.float32),
        pltpu.VMEM_SHARED((_NHT, 16), jnp.float32),
        pltpu.SemaphoreType.DMA,
        pltpu.SemaphoreType.DMA,
        pltpu.SemaphoreType.DMA,
        pltpu.SemaphoreType.DMA,
    ]

    @functools.partial(
        pl.kernel, mesh=mesh, out_type=out_type, scratch_types=scratch_types,
        compiler_params=pltpu.CompilerParams(needs_layout_passes=False,
                                             use_tc_tiling_on_sc=False))
    def edge_kernel(src_hbm, dst_hbm, wh_hbm, s_hbm, d_hbm, m_hbm,
                    num_out, den_out,
                    src_all, dst_all, idx2_all, idxd_all,
                    s_a, s_b, d_a, d_b, ex_a, ex_b, wh_a, wh_b,
                    m_v, zb, zbd, num_sh, den_sh, gs_a, gs_b, ss_a, ss_b):
        c = lax.axis_index("c")
        sid = lax.axis_index("s")
        zvec = jnp.zeros((16,), jnp.float32)

        def zrow(r, carry):
            for g in range(groups):
                zb[r, pl.ds(g * 16, 16)] = zvec
            zbd[r, :] = zvec
            return carry
        lax.fori_loop(0, _ZR, zrow, 0)

        r0 = sid * _RPT
        rd0 = sid * _RPTD
        w = sid * _NCORES + c
        start = w * _TPW
        cps = []
        for k in range(_RPT // _ZR):
            cps.append(pltpu.async_copy(
                zb, num_sh.at[pl.ds(r0 + k * _ZR, _ZR)], gs_a))
        for k in range(_RPTD // _ZR):
            cps.append(pltpu.async_copy(
                zbd, den_sh.at[pl.ds(rd0 + k * _ZR, _ZR)], gs_b))
        cps.append(pltpu.async_copy(
            zbd.at[pl.ds(0, 8)], den_sh.at[pl.ds(_NH + sid * 8, 8)], gs_b))
        cps.append(pltpu.async_copy(m_hbm, m_v, ss_a))
        cps.append(pltpu.async_copy(
            src_hbm.at[pl.ds(start, _TPW)], src_all, ss_a))
        cps.append(pltpu.async_copy(
            dst_hbm.at[pl.ds(start, _TPW)], dst_all, ss_a))
        for cp in cps:
            cp.wait()
        plsc.subcore_barrier()

        msum = m_v[0, :] + m_v[1, :]
        mvec = jnp.maximum(msum, 0.2 * msum)
        lanes = jax.lax.iota(jnp.int32, 16)
        trash = lanes % 8 + (_NH + sid * 8)
        coff = c * _N
        dlo = c * _NH

        def prep(row, carry):
            for j in range(_C // 16):
                sl = pl.ds(j * 16, 16)
                idx2_all[row, sl] = src_all[row, sl] + coff
                t = dst_all[row, sl] - dlo
                oob = (t < 0) | (t >= _NH)
                idxd_all[row, sl] = jnp.where(oob, trash, t)
            return carry
        lax.fori_loop(0, _TPW, prep, 0)

        def fire_g(sv, dv, whv, gsem, k):
            pltpu.async_copy(s_hbm.at[src_all.at[k]], sv, gsem)
            pltpu.async_copy(d_hbm.at[dst_all.at[k]], dv, gsem)
            pltpu.async_copy(wh_hbm.at[idx2_all.at[k]], whv, gsem)

        def drain_g(sv, dv, whv, gsem):
            pltpu.make_async_copy(s_hbm.at[pl.ds(0, _C)], sv, gsem).wait()
            pltpu.make_async_copy(d_hbm.at[pl.ds(0, _C)], dv, gsem).wait()
            pltpu.make_async_copy(wh_hbm.at[pl.ds(0, _C)], whv, gsem).wait()

        def fire_s(whv, exv, ssem, k):
            pltpu.async_copy(whv, num_sh.at[dst_all.at[k]], ssem, add=True)
            if q == 0:
                pltpu.async_copy(exv, den_sh.at[idxd_all.at[k]], ssem,
                                 add=True)

        def drain_s(ssem):
            pltpu.make_async_copy(
                wh_hbm.at[pl.ds(0, _C)], num_sh.at[pl.ds(0, _C)], ssem).wait()
            if q == 0:
                pltpu.make_async_copy(
                    s_hbm.at[pl.ds(0, _C)], den_sh.at[pl.ds(0, _C)],
                    ssem).wait()

        def compute(s_rows, d_rows, ex_v, wh_v):
            def exrow(r4, carry2):
                rs = [r4 * 4 + u for u in range(4)]
                svs = [s_rows[r, :] for r in rs]
                dvs = [d_rows[r, :] for r in rs]
                es = [s + d for s, d in zip(svs, dvs)]
                lrs = [jnp.maximum(e, 0.2 * e) for e in es]
                exs = [jnp.exp(lr - mvec) for lr in lrs]
                for r, ex in zip(rs, exs):
                    ex_v[r, :] = ex
                return carry2
            lax.fori_loop(0, _C // 4, exrow, 0)

            def wloop(glanes):
                def wrow(r2, carry2):
                    for u in range(2):
                        r = r2 * 2 + u
                        row16 = ex_v[r, :]
                        wv = {}
                        for g in range(groups):
                            h = glanes[g]
                            if h not in wv:
                                wv[h] = lax.squeeze(
                                    lax.slice_in_dim(row16, h, h + 1, axis=0),
                                    dimensions=[0])
                            wh_v[r, pl.ds(g * 16, 16)] = (
                                wh_v[r, pl.ds(g * 16, 16)] * wv[h])
                    return carry2
                lax.fori_loop(0, _C // 2, wrow, 0)

            @pl.when(c == 0)
            def _():
                wloop(lanes0)

            @pl.when(c == 1)
            def _():
                wloop(lanes1)

        fire_g(s_a, d_a, wh_a, gs_a, 0)

        def pair(j, carry):
            drain_g(s_a, d_a, wh_a, gs_a)

            @pl.when(j > 0)
            def _():
                drain_s(ss_b)
            fire_g(s_b, d_b, wh_b, gs_b, 2 * j + 1)
            compute(s_a, d_a, ex_a, wh_a)
            fire_s(wh_a, ex_a, ss_a, 2 * j)
            drain_g(s_b, d_b, wh_b, gs_b)
            compute(s_b, d_b, ex_b, wh_b)
            drain_s(ss_a)

            @pl.when(j < _PAIRS - 1)
            def _():
                fire_g(s_a, d_a, wh_a, gs_a, 2 * j + 2)
            fire_s(wh_b, ex_b, ss_b, 2 * j + 1)
            return carry
        lax.fori_loop(0, _PAIRS, pair, 0)
        drain_s(ss_b)

        plsc.subcore_barrier()
        wb1 = pltpu.async_copy(num_sh.at[pl.ds(r0, _RPT)],
                               num_out.at[c, pl.ds(r0, _RPT)], gs_a)
        wb2 = pltpu.async_copy(den_sh.at[pl.ds(rd0, _RPTD)],
                               den_out.at[pl.ds(c * _NH + rd0, _RPTD)], gs_b)
        wb1.wait()
        wb2.wait()

    return edge_kernel


_edge1 = _make_edge_kernel(48, 16, 0)
_edge2 = _make_edge_kernel(96, 32, 0)
_edge3a = _make_edge_kernel(96, 64, 0)
_edge3b = _make_edge_kernel(96, 64, 1)


def _elu(x):
    return jnp.where(x > 0, x, jnp.exp(jnp.minimum(x, 0.0)) - 1.0)


def _dense_common(i, h, w_ref, as_ref, ad_ref, wh_ref, s_ref, d_ref, m_ref,
                  nparts):
    wh = jnp.dot(h, w_ref[...], preferred_element_type=jnp.float32)
    s16 = jnp.dot(wh, as_ref[...], preferred_element_type=jnp.float32)
    d16 = jnp.dot(wh, ad_ref[...], preferred_element_type=jnp.float32)
    part = wh.shape[1] // nparts
    for j in range(nparts):
        wh_ref[j] = wh[:, j * part:(j + 1) * part]
    s_ref[...] = s16
    d_ref[...] = d16
    cur = jnp.concatenate(
        [jnp.max(s16, axis=0, keepdims=True),
         jnp.max(d16, axis=0, keepdims=True)], axis=0)

    @pl.when(i == 0)
    def _():
        m_ref[...] = cur

    @pl.when(i > 0)
    def _():
        m_ref[...] = jnp.maximum(m_ref[...], cur)


def _dense_outs(hfn, nparts):
    part = hfn // nparts
    out_specs = [
        pl.BlockSpec((nparts, _BN, part), lambda i: (0, i, 0)),
        pl.BlockSpec((_BN, 16), lambda i: (i, 0)),
        pl.BlockSpec((_BN, 16), lambda i: (i, 0)),
        pl.BlockSpec((2, 16), lambda i: (0, 0)),
    ]
    out_shape = [
        jax.ShapeDtypeStruct((nparts, _N, part), jnp.float32),
        jax.ShapeDtypeStruct((_N, 16), jnp.float32),
        jax.ShapeDtypeStruct((_N, 16), jnp.float32),
        jax.ShapeDtypeStruct((2, 16), jnp.float32),
    ]
    return out_specs, out_shape


def _make_dense1(hfn, nparts):
    def body(x_ref, w_ref, as_ref, ad_ref, wh_ref, s_ref, d_ref, m_ref):
        i = pl.program_id(0)
        _dense_common(i, x_ref[...], w_ref, as_ref, ad_ref,
                      wh_ref, s_ref, d_ref, m_ref, nparts)

    din = 11
    out_specs, out_shape = _dense_outs(hfn, nparts)
    return pl.pallas_call(
        body,
        grid=(_NBLK,),
        in_specs=[
            pl.BlockSpec((_BN, din), lambda i: (i, 0)),
            pl.BlockSpec((din, hfn), lambda i: (0, 0)),
            pl.BlockSpec((hfn, 16), lambda i: (0, 0)),
            pl.BlockSpec((hfn, 16), lambda i: (0, 0)),
        ],
        out_specs=out_specs,
        out_shape=out_shape,
    )


def _finalize(num_refs, den_ref, e_ref):
    cat = jnp.concatenate([r[j] for r in num_refs for j in (0, 1)], axis=1)
    dexp = jnp.dot(den_ref[...], e_ref[...], preferred_element_type=jnp.float32)
    return _elu(cat / (dexp + 1e-16))


def _make_dense(hfp, hfn, nin, nparts):
    part_p = hfp // (2 * nin)

    def body(*refs):
        num_refs = refs[:nin]
        den_ref, e_ref, w_ref, as_ref, ad_ref = refs[nin:nin + 5]
        wh_ref, s_ref, d_ref, m_ref = refs[nin + 5:]
        i = pl.program_id(0)
        h = _finalize(num_refs, den_ref, e_ref)
        _dense_common(i, h, w_ref, as_ref, ad_ref,
                      wh_ref, s_ref, d_ref, m_ref, nparts)

    out_specs, out_shape = _dense_outs(hfn, nparts)
    return pl.pallas_call(
        body,
        grid=(_NBLK,),
        in_specs=[
            pl.BlockSpec((2, _BN, part_p), lambda i: (0, i, 0))
            for _ in range(nin)
        ] + [
            pl.BlockSpec((_BN, 16), lambda i: (i, 0)),
            pl.BlockSpec((16, hfp), lambda i: (0, 0)),
            pl.BlockSpec((hfp, hfn), lambda i: (0, 0)),
            pl.BlockSpec((hfn, 16), lambda i: (0, 0)),
            pl.BlockSpec((hfn, 16), lambda i: (0, 0)),
        ],
        out_specs=out_specs,
        out_shape=out_shape,
    )


def _make_readout(hfp, nin):
    part_p = hfp // (2 * nin)

    def body(*refs):
        ni_refs = refs[:nin]
        di_ref = refs[nin]
        nn_refs = refs[nin + 1:2 * nin + 1]
        dn_ref, e_ref, wd_ref, bd_ref = refs[2 * nin + 1:2 * nin + 5]
        acc_ref, out_ref = refs[2 * nin + 5:]
        i = pl.program_id(0)
        hi = _finalize(ni_refs, di_ref, e_ref)
        hn = _finalize(nn_refs, dn_ref, e_ref)
        both = jnp.concatenate([hi, hn], axis=1)
        ps = jnp.sum(both, axis=0, keepdims=True)

        @pl.when(i == 0)
        def _():
            acc_ref[...] = ps

        @pl.when(i > 0)
        def _():
            acc_ref[...] = acc_ref[...] + ps

        @pl.when(i == _NBLK - 1)
        def _():
            s = acc_ref[...]
            nrm = jnp.maximum(jnp.sqrt(jnp.sum(s * s)), 1e-12)
            out_ref[...] = (jnp.dot(s / nrm, wd_ref[...],
                                    preferred_element_type=jnp.float32)
                            + bd_ref[...])

    num_spec = pl.BlockSpec((2, _BN, part_p), lambda i: (0, i, 0))
    den_spec = pl.BlockSpec((_BN, 16), lambda i: (i, 0))
    return pl.pallas_call(
        body,
        grid=(_NBLK,),
        in_specs=[num_spec] * nin + [den_spec] + [num_spec] * nin + [
            den_spec,
            pl.BlockSpec((16, hfp), lambda i: (0, 0)),
            pl.BlockSpec((2 * hfp, 1), lambda i: (0, 0)),
            pl.BlockSpec((1, 1), lambda i: (0, 0)),
        ],
        out_specs=[
            pl.BlockSpec((1, 2 * hfp), lambda i: (0, 0)),
            pl.BlockSpec((1, 1), lambda i: (0, 0)),
        ],
        out_shape=[
            jax.ShapeDtypeStruct((1, 2 * hfp), jnp.float32),
            jax.ShapeDtypeStruct((1, 1), jnp.float32),
        ],
    )


_dense1 = _make_dense1(96, 2)
_dense2 = _make_dense(96, 192, 1, 2)
_dense3 = _make_dense(192, 384, 1, 4)
_readout = _make_readout(384, 2)


def _blockdiag16(a):
    h, f = a.shape
    rows = jnp.arange(h * f)
    heads = rows // f
    flat = a.reshape(-1)
    out = jnp.zeros((h * f, 16), jnp.float32)
    out = out.at[rows, heads].set(flat)
    out = out.at[rows, heads + 8].set(flat)
    return out


def _expand16(hf, f):
    cols = jnp.arange(hf)
    heads = cols // f
    return jnp.zeros((16, hf), jnp.float32).at[heads, cols].set(1.0)


def kernel(x, edge_index_int, edge_index_nh, W1, a1_src, a1_dst,
           W2, a2_src, a2_dst, W3, a3_src, a3_dst, Wd, bd):
    a1s, a1d = _blockdiag16(a1_src), _blockdiag16(a1_dst)
    a2s, a2d = _blockdiag16(a2_src), _blockdiag16(a2_dst)
    a3s, a3d = _blockdiag16(a3_src), _blockdiag16(a3_dst)
    e1 = _expand16(96, 16)
    e2 = _expand16(192, 32)
    e3 = _expand16(384, 64)

    pad_src = jnp.arange(_EP - _E, dtype=jnp.int32) % _N
    pad_dst = jnp.arange(_EP - _E, dtype=jnp.int32) % (_NPAD - _N) + _N

    def pad2d(sv, dv):
        s2 = jnp.concatenate([sv, pad_src]).reshape(_NCHP, _C)
        d2 = jnp.concatenate([dv, pad_dst]).reshape(_NCHP, _C)
        return s2, d2

    si, di_ = pad2d(edge_index_int[0], edge_index_int[1])
    sn, dn_ = pad2d(edge_index_nh[0], edge_index_nh[1])

    wh1, s1, d1, m1 = _dense1(x, W1, a1s, a1d)
    wh1f = wh1.reshape(2 * _N, 48)
    ni1, den_i1 = _edge1(si, di_, wh1f, s1, d1, m1)
    nn1, den_n1 = _edge1(sn, dn_, wh1f, s1, d1, m1)

    wh2i, s2i, d2i, m2i = _dense2(ni1, den_i1, e1, W2, a2s, a2d)
    wh2n, s2n, d2n, m2n = _dense2(nn1, den_n1, e1, W2, a2s, a2d)
    ni2, den_i2 = _edge2(si, di_, wh2i.reshape(2 * _N, 96), s2i, d2i, m2i)
    nn2, den_n2 = _edge2(sn, dn_, wh2n.reshape(2 * _N, 96), s2n, d2n, m2n)

    wh3i, s3i, d3i, m3i = _dense3(ni2, den_i2, e2, W3, a3s, a3d)
    wh3n, s3n, d3n, m3n = _dense3(nn2, den_n2, e2, W3, a3s, a3d)
    wh3i4 = wh3i.reshape(4 * _N, 96)
    wh3n4 = wh3n.reshape(4 * _N, 96)
    ni3a, den_i3 = _edge3a(si, di_, wh3i4[:2 * _N], s3i, d3i, m3i)
    ni3b, _ = _edge3b(si, di_, wh3i4[2 * _N:], s3i, d3i, m3i)
    nn3a, den_n3 = _edge3a(sn, dn_, wh3n4[:2 * _N], s3n, d3n, m3n)
    nn3b, _ = _edge3b(sn, dn_, wh3n4[2 * _N:], s3n, d3n, m3n)

    _, out = _readout(ni3a, ni3b, den_i3, nn3a, nn3b, den_n3,
                      e3, Wd, bd.reshape(1, 1))
    return out.reshape((1,))
